# Initial kernel scaffold; baseline (speedup 1.0000x reference)
import jax
import jax.numpy as jnp
from jax.experimental import pallas as pl


def kernel(agents, passengers, vectors):
    raise NotImplementedError("write your pallas kernel here")


# speedup vs baseline: 11.2683x; 11.2683x over previous
import functools

import jax
import jax.numpy as jnp
from jax import lax
from jax.experimental import pallas as pl
from jax.experimental.pallas import tpu as pltpu
from jax.experimental.pallas import tpu_sc as plsc

E = 1024
A = 128
P = 400000


def _tc_body(vt_ref, at_ref, na_ref, dist_ref, tab_ref):
    sx = vt_ref[0 * E:1 * E, :]
    sy = vt_ref[1 * E:2 * E, :]
    gx = vt_ref[2 * E:3 * E, :]
    gy = vt_ref[3 * E:4 * E, :]
    bx = jnp.clip(gx - sx, -1, 1)
    by = jnp.clip(gy - sy, -1, 1)
    bx = jnp.where(sx == -100, 0, bx)
    by = jnp.where(sy == -100, 0, by)
    na_ref[0 * E:1 * E, :] = at_ref[0 * E:1 * E, :] + bx
    na_ref[1 * E:2 * E, :] = at_ref[1 * E:2 * E, :] + by
    dist_ref[...] = jnp.sqrt((bx * bx + by * by).astype(jnp.float32))
    e = ((by + 1) << 2) | (bx + 1)
    acc = e[0 * A:1 * A, :]
    for j in range(1, 8):
        acc = acc | (e[j * A:(j + 1) * A, :] << (4 * j))
    tab_ref[...] = acc


_tc_call = pl.pallas_call(
    _tc_body,
    out_shape=(
        jax.ShapeDtypeStruct((2 * E, A), jnp.int32),
        jax.ShapeDtypeStruct((E, A), jnp.float32),
        jax.ShapeDtypeStruct((A, A), jnp.int32),
    ),
)


_NC, _NS, _L = 2, 16, 16
_NW = _NC * _NS
_RPW = P // _NW
_CHUNK = 2500
_NCHUNK = _RPW // _CHUNK
_FULL_G = _CHUNK // _L
_TAIL = _CHUNK - _FULL_G * _L
_CW = _CHUNK * 8
_TABW = (E * A) // 8


def _sc_body(pflat_hbm, tab_hbm, out_hbm, tab_v, chunk_v):
    wid = lax.axis_index("s") * _NC + lax.axis_index("c")
    pltpu.sync_copy(tab_hbm, tab_v)
    iota = lax.iota(jnp.int32, _L)

    def group(rid, mask):
        b = rid * 8
        p0 = plsc.load_gather(chunk_v, [b])
        p6 = plsc.load_gather(chunk_v, [b + 6])
        p7 = plsc.load_gather(chunk_v, [b + 7])
        g = p0 * A + p7
        w = plsc.load_gather(tab_v, [g & (_TABW - 1)])
        e = (w >> ((g >> 14) << 2)) & 15
        move = (p6 != 0) & (p6 != 1)
        bx = jnp.where(move, (e & 3) - 1, 0)
        by = jnp.where(move, (e >> 2) - 1, 0)
        plsc.addupdate_scatter(chunk_v, [b + 1], bx, mask=mask)
        plsc.addupdate_scatter(chunk_v, [b + 2], by, mask=mask)

    def do_chunk(c, carry):
        base = (wid * _RPW + c * _CHUNK) * 8
        pltpu.sync_copy(pflat_hbm.at[pl.ds(base, _CW)], chunk_v)

        def body(gi, carry2):
            group(gi * _L + iota, None)
            return carry2

        lax.fori_loop(0, _FULL_G, body, 0)
        rid = jnp.minimum(_FULL_G * _L + iota, _CHUNK - 1)
        group(rid, iota < _TAIL)
        pltpu.sync_copy(chunk_v, out_hbm.at[pl.ds(base, _CW)])
        return carry

    lax.fori_loop(0, _NCHUNK, do_chunk, 0)


@functools.cache
def _sc_call():
    return functools.partial(
        pl.kernel,
        out_type=jax.ShapeDtypeStruct((P * 8,), jnp.int32),
        mesh=plsc.VectorSubcoreMesh(
            core_axis_name="c", subcore_axis_name="s",
            num_cores=_NC, num_subcores=_NS),
        scratch_types=[
            pltpu.VMEM((_TABW,), jnp.int32),
            pltpu.VMEM((_CW,), jnp.int32),
        ],
        compiler_params=pltpu.CompilerParams(needs_layout_passes=False),
    )(_sc_body)


def kernel(agents, passengers, vectors):
    vt = vectors.transpose(2, 0, 1).reshape(4 * E, A)
    at = agents.transpose(2, 0, 1).reshape(2 * E, A)
    na_t, distances, tab = _tc_call(vt, at)
    new_agents = na_t.reshape(2, E, A).transpose(1, 2, 0)
    out = _sc_call()(passengers.reshape(-1), tab.reshape(-1))
    new_passengers = out.reshape(P, 8)
    return new_agents, new_passengers, distances


# speedup vs baseline: 59.6194x; 5.2909x over previous
import functools

import jax
import jax.numpy as jnp
from jax import lax
from jax.experimental import pallas as pl
from jax.experimental.pallas import tpu as pltpu
from jax.experimental.pallas import tpu_sc as plsc

E = 1024
A = 128
P = 400000


def _tc_body(vt_ref, at_ref, na_ref, dist_ref, tab_ref):
    sx = vt_ref[0 * E:1 * E, :]
    sy = vt_ref[1 * E:2 * E, :]
    gx = vt_ref[2 * E:3 * E, :]
    gy = vt_ref[3 * E:4 * E, :]
    bx = jnp.clip(gx - sx, -1, 1)
    by = jnp.clip(gy - sy, -1, 1)
    bx = jnp.where(sx == -100, 0, bx)
    by = jnp.where(sy == -100, 0, by)
    na_ref[0 * E:1 * E, :] = at_ref[0 * E:1 * E, :] + bx
    na_ref[1 * E:2 * E, :] = at_ref[1 * E:2 * E, :] + by
    dist_ref[...] = jnp.sqrt((bx * bx + by * by).astype(jnp.float32))
    e = ((by + 1) << 2) | (bx + 1)
    acc = e[0 * A:1 * A, :]
    for j in range(1, 8):
        acc = acc | (e[j * A:(j + 1) * A, :] << (4 * j))
    tab_ref[...] = acc


_tc_call = pl.pallas_call(
    _tc_body,
    out_shape=(
        jax.ShapeDtypeStruct((2 * E, A), jnp.int32),
        jax.ShapeDtypeStruct((E, A), jnp.float32),
        jax.ShapeDtypeStruct((A, A), jnp.int32),
    ),
)


_NC, _NS, _L = 2, 16, 16
_NW = _NC * _NS
_SPAN_LO, _SPAN_HI = 12496, 12504
_GROUPS = 391
_CPAD = _GROUPS * _L
_TABW = (E * A) // 8


def _sc_body(pt_hbm, tab_hbm, out_hbm, tab_v, *cols):
    wid = lax.axis_index("s") * _NC + lax.axis_index("c")
    pltpu.sync_copy(tab_hbm, tab_v)

    def group(gi, carry):
        s = pl.ds(gi * _L, _L)
        p0 = cols[0][s]
        p6 = cols[6][s]
        p7 = cols[7][s]
        g = p0 * A + p7
        w = plsc.load_gather(tab_v, [g & (_TABW - 1)])
        e = (w >> (((g >> 14) & 7) << 2)) & 15
        move = (p6 != 0) & (p6 != 1)
        bx = jnp.where(move, (e & 3) - 1, 0)
        by = jnp.where(move, (e >> 2) - 1, 0)
        cols[1][s] = cols[1][s] + bx
        cols[2][s] = cols[2][s] + by
        return carry

    def do_chunk(base, n):
        for col in range(8):
            pltpu.sync_copy(pt_hbm.at[pl.ds(col * P + base, n)],
                            cols[col].at[pl.ds(0, n)])
        lax.fori_loop(0, _GROUPS, group, 0)
        for col in range(8):
            pltpu.sync_copy(cols[col].at[pl.ds(0, n)],
                            out_hbm.at[pl.ds(col * P + base, n)])

    @pl.when(wid < 16)
    def _():
        base = wid * _SPAN_LO
        do_chunk(base, 6248)
        do_chunk(base + 6248, 6248)

    @pl.when(wid >= 16)
    def _():
        base = 16 * _SPAN_LO + (wid - 16) * _SPAN_HI
        do_chunk(base, 6248)
        do_chunk(base + 6248, 6256)


@functools.cache
def _sc_call():
    return functools.partial(
        pl.kernel,
        out_type=jax.ShapeDtypeStruct((8 * P,), jnp.int32),
        mesh=plsc.VectorSubcoreMesh(
            core_axis_name="c", subcore_axis_name="s",
            num_cores=_NC, num_subcores=_NS),
        scratch_types=[pltpu.VMEM((_TABW,), jnp.int32)]
        + [pltpu.VMEM((_CPAD,), jnp.int32) for _ in range(8)],
        compiler_params=pltpu.CompilerParams(needs_layout_passes=False),
    )(_sc_body)


def kernel(agents, passengers, vectors):
    vt = vectors.transpose(2, 0, 1).reshape(4 * E, A)
    at = agents.transpose(2, 0, 1).reshape(2 * E, A)
    na_t, distances, tab = _tc_call(vt, at)
    new_agents = na_t.reshape(2, E, A).transpose(1, 2, 0)
    out = _sc_call()(passengers.T.reshape(-1), tab.reshape(-1))
    new_passengers = out.reshape(8, P).T
    return new_agents, new_passengers, distances
